# baseline (speedup 1.0000x reference)
import functools
import jax
import jax.numpy as jnp
from jax import lax
from jax.experimental import pallas as pl
from jax.experimental.pallas import tpu as pltpu
from jax.experimental.pallas import tpu_sc as plsc

B = 1024
T = 200
NPH = 1001
NSG = 1000
NLG = 1000
PH_DIM = 128
SG_DIM = 16
LG_DIM = 8
NMEL = 80

NW = 32
BPW = B // NW
LANES = 16
UNROLL = 8


def _tc_precompute(pt_ref, st_ref, lt_ref, sid_ref, lid_ref, w_ref, bias_ref,
                   p_ref, base_ref):
    W = w_ref[...]
    hp = lax.Precision.HIGHEST
    p_ref[...] = jnp.dot(pt_ref[...], W[1:1 + PH_DIM],
                         preferred_element_type=jnp.float32, precision=hp)
    SW = jnp.dot(st_ref[...], W[1 + PH_DIM:1 + PH_DIM + SG_DIM],
                 preferred_element_type=jnp.float32, precision=hp)
    LW = jnp.dot(lt_ref[...], W[1 + PH_DIM + SG_DIM:],
                 preferred_element_type=jnp.float32, precision=hp)
    iota_s = lax.broadcasted_iota(jnp.int32, (B, NSG), 1)
    oh_s = (sid_ref[...] == iota_s).astype(jnp.float32)
    oh_l = (lid_ref[...] == iota_s).astype(jnp.float32)
    base = (jnp.dot(oh_s, SW, preferred_element_type=jnp.float32, precision=hp)
            + jnp.dot(oh_l, LW, preferred_element_type=jnp.float32, precision=hp)
            + bias_ref[...])
    base_ref[...] = base


def _precompute(phoneme_table, singer_table, language_table, sid, lid, W, bias):
    return pl.pallas_call(
        _tc_precompute,
        out_shape=[
            jax.ShapeDtypeStruct((NPH, NMEL), jnp.float32),
            jax.ShapeDtypeStruct((B, NMEL), jnp.float32),
        ],
    )(phoneme_table, singer_table, language_table, sid, lid, W, bias)


def _sc_body(p_hbm, w_hbm, base_hbm, f0_hbm, idx_hbm, out_hbm,
             p_loc, w0_loc, base_loc, f0a, idxa, st0, st1, osem0, osem1):
    stages = (st0, st1)
    wid = lax.axis_index("s") * 2 + lax.axis_index("c")
    b0 = wid * BPW

    pltpu.sync_copy(p_hbm, p_loc)
    pltpu.sync_copy(w_hbm.at[0], w0_loc)
    pltpu.sync_copy(base_hbm.at[pl.ds(b0 * NMEL, BPW * NMEL)], base_loc)
    pltpu.sync_copy(f0_hbm.at[wid], f0a)
    pltpu.sync_copy(idx_hbm.at[wid], idxa)

    iotav = lax.iota(jnp.int32, LANES)
    w0v = [w0_loc[pl.ds(16 * k, 16)] for k in range(5)]

    def fill(bl, buf):
        basev = [base_loc[pl.ds(bl * NMEL + 16 * k, 16)] for k in range(5)]
        tok0 = bl * T

        @plsc.parallel_loop(0, T, unroll=UNROLL)
        def tok_body(t):
            ts = jnp.full((LANES,), tok0 + t, dtype=jnp.int32)
            r = plsc.load_gather(idxa, [ts])
            f = plsc.load_gather(f0a, [ts])
            rbase = r * NMEL
            for k in range(5):
                g5 = plsc.load_gather(p_loc, [rbase + (iotav + 16 * k)])
                stages[buf][pl.ds(t * NMEL + 16 * k, 16)] = (
                    g5 + (f * w0v[k] + basev[k]))

    RWT = T * NMEL
    HW = RWT // 2

    def copy_out(b, buf):
        st = stages[buf]
        pltpu.async_copy(st.at[pl.ds(0, HW)],
                         out_hbm.at[pl.ds(b * RWT, HW)], osem0)
        pltpu.async_copy(st.at[pl.ds(HW, HW)],
                         out_hbm.at[pl.ds(b * RWT + HW, HW)], osem1)

    def wait_out(b, buf):
        st = stages[buf]
        pltpu.make_async_copy(st.at[pl.ds(0, HW)],
                              out_hbm.at[pl.ds(b * RWT, HW)], osem0).wait()
        pltpu.make_async_copy(st.at[pl.ds(HW, HW)],
                              out_hbm.at[pl.ds(b * RWT + HW, HW)], osem1).wait()

    fill(0, 0)
    copy_out(b0, 0)
    fill(1, 1)
    copy_out(b0 + 1, 1)

    def pair_body(i, c):
        b = b0 + 2 * i
        wait_out(b, 0)
        fill(2 * i, 0)
        copy_out(b, 0)
        wait_out(b + 1, 1)
        fill(2 * i + 1, 1)
        copy_out(b + 1, 1)
        return c

    lax.fori_loop(1, BPW // 2, pair_body, 0)
    wait_out(b0, 0)
    wait_out(b0 + 1, 1)


@functools.lru_cache(maxsize=1)
def _sc_lookup():
    mesh = plsc.VectorSubcoreMesh(core_axis_name="c", subcore_axis_name="s")
    return pl.kernel(
        _sc_body,
        out_type=jax.ShapeDtypeStruct((B * T * NMEL,), jnp.float32),
        mesh=mesh,
        compiler_params=pltpu.CompilerParams(needs_layout_passes=False),
        scratch_types=[
            pltpu.VMEM((NPH * NMEL,), jnp.float32),
            pltpu.VMEM((NMEL,), jnp.float32),
            pltpu.VMEM((BPW * NMEL,), jnp.float32),
            pltpu.VMEM((BPW * T,), jnp.float32),
            pltpu.VMEM((BPW * T,), jnp.int32),
            pltpu.VMEM((T * NMEL,), jnp.float32),
            pltpu.VMEM((T * NMEL,), jnp.float32),
            pltpu.SemaphoreType.DMA,
            pltpu.SemaphoreType.DMA,
        ],
    )


def kernel(f0, phoneme_seq, singer_id, language_id, phoneme_table,
           singer_table, language_table, W, b):
    idx = phoneme_seq.astype(jnp.int32)
    sid = singer_id.astype(jnp.int32).reshape(B, 1)
    lid = language_id.astype(jnp.int32).reshape(B, 1)
    bias = b.reshape(1, NMEL)

    P, base = _precompute(phoneme_table, singer_table, language_table,
                          sid, lid, W, bias)

    out = _sc_lookup()(P.reshape(-1), W, base.reshape(-1),
                       f0.reshape(NW, BPW * T), idx.reshape(NW, BPW * T))
    return out.reshape(B, T, NMEL)


# speedup vs baseline: 1.5150x; 1.5150x over previous
import functools
import jax
import jax.numpy as jnp
from jax import lax
from jax.experimental import pallas as pl
from jax.experimental.pallas import tpu as pltpu
from jax.experimental.pallas import tpu_sc as plsc

B = 1024
T = 200
NPH = 1001
NSG = 1000
NLG = 1000
PH_DIM = 128
SG_DIM = 16
LG_DIM = 8
NMEL = 80

NW = 32
BPW = B // NW
LANES = 16
UNROLL = 8


def _tc_precompute(pt_ref, st_ref, lt_ref, sid_ref, lid_ref, w_ref, bias_ref,
                   p_ref, base_ref):
    W = w_ref[...]
    hp = lax.Precision.HIGHEST
    p_ref[...] = jnp.dot(pt_ref[...], W[1:1 + PH_DIM],
                         preferred_element_type=jnp.float32, precision=hp)
    SW = jnp.dot(st_ref[...], W[1 + PH_DIM:1 + PH_DIM + SG_DIM],
                 preferred_element_type=jnp.float32, precision=hp)
    LW = jnp.dot(lt_ref[...], W[1 + PH_DIM + SG_DIM:],
                 preferred_element_type=jnp.float32, precision=hp)
    iota_s = lax.broadcasted_iota(jnp.int32, (B, NSG), 1)
    oh_s = (sid_ref[...] == iota_s).astype(jnp.float32)
    oh_l = (lid_ref[...] == iota_s).astype(jnp.float32)
    base = (jnp.dot(oh_s, SW, preferred_element_type=jnp.float32, precision=hp)
            + jnp.dot(oh_l, LW, preferred_element_type=jnp.float32, precision=hp)
            + bias_ref[...])
    base_ref[...] = base


def _precompute(phoneme_table, singer_table, language_table, sid, lid, W, bias):
    return pl.pallas_call(
        _tc_precompute,
        out_shape=[
            jax.ShapeDtypeStruct((NPH, NMEL), jnp.float32),
            jax.ShapeDtypeStruct((B, NMEL), jnp.float32),
        ],
    )(phoneme_table, singer_table, language_table, sid, lid, W, bias)


def _sc_body(p_hbm, w_hbm, base_hbm, f0_hbm, idx_hbm, out_hbm,
             p_loc, w0_loc, base_loc, f0a, idxa, out_stage, osem0, osem1):
    wid = lax.axis_index("s") * 2 + lax.axis_index("c")
    b0 = wid * BPW

    PH = NPH * NMEL // 2
    c0 = pltpu.async_copy(p_hbm.at[pl.ds(0, PH)], p_loc.at[pl.ds(0, PH)], osem0)
    c1 = pltpu.async_copy(p_hbm.at[pl.ds(PH, PH)], p_loc.at[pl.ds(PH, PH)],
                          osem1)
    c2 = pltpu.async_copy(w_hbm.at[0], w0_loc, osem0)
    c3 = pltpu.async_copy(base_hbm.at[pl.ds(b0 * NMEL, BPW * NMEL)], base_loc,
                          osem1)
    c4 = pltpu.async_copy(f0_hbm.at[wid], f0a, osem0)
    c5 = pltpu.async_copy(idx_hbm.at[wid], idxa, osem1)
    for c in (c0, c1, c2, c3, c4, c5):
        c.wait()

    iotav = lax.iota(jnp.int32, LANES)
    w0v = [w0_loc[pl.ds(16 * k, 16)] for k in range(5)]

    def fill(bl, buf):
        basev = [base_loc[pl.ds(bl * NMEL + 16 * k, 16)] for k in range(5)]
        tok0 = bl * T

        @plsc.parallel_loop(0, T, unroll=UNROLL)
        def tok_body(t):
            ts = jnp.full((LANES,), tok0 + t, dtype=jnp.int32)
            r = plsc.load_gather(idxa, [ts])
            f = plsc.load_gather(f0a, [ts])
            rbase = r * NMEL
            for k in range(5):
                g5 = plsc.load_gather(p_loc, [rbase + (iotav + 16 * k)])
                out_stage[buf, pl.ds(t * NMEL + 16 * k, 16)] = (
                    g5 + (f * w0v[k] + basev[k]))

    fill(0, 0)
    pltpu.async_copy(out_stage.at[0], out_hbm.at[b0], osem0)
    fill(1, 1)
    pltpu.async_copy(out_stage.at[1], out_hbm.at[b0 + 1], osem1)

    def pair_body(i, c):
        b = b0 + 2 * i
        pltpu.make_async_copy(out_stage.at[0], out_hbm.at[b], osem0).wait()
        fill(2 * i, 0)
        pltpu.async_copy(out_stage.at[0], out_hbm.at[b], osem0)
        pltpu.make_async_copy(out_stage.at[1], out_hbm.at[b + 1], osem1).wait()
        fill(2 * i + 1, 1)
        pltpu.async_copy(out_stage.at[1], out_hbm.at[b + 1], osem1)
        return c

    lax.fori_loop(1, BPW // 2, pair_body, 0)
    pltpu.make_async_copy(out_stage.at[0], out_hbm.at[b0], osem0).wait()
    pltpu.make_async_copy(out_stage.at[1], out_hbm.at[b0 + 1], osem1).wait()


@functools.lru_cache(maxsize=1)
def _sc_lookup():
    mesh = plsc.VectorSubcoreMesh(core_axis_name="c", subcore_axis_name="s")
    return pl.kernel(
        _sc_body,
        out_type=jax.ShapeDtypeStruct((B, T * NMEL), jnp.float32),
        mesh=mesh,
        compiler_params=pltpu.CompilerParams(needs_layout_passes=False),
        scratch_types=[
            pltpu.VMEM((NPH * NMEL,), jnp.float32),
            pltpu.VMEM((NMEL,), jnp.float32),
            pltpu.VMEM((BPW * NMEL,), jnp.float32),
            pltpu.VMEM((BPW * T,), jnp.float32),
            pltpu.VMEM((BPW * T,), jnp.int32),
            pltpu.VMEM((2, T * NMEL), jnp.float32),
            pltpu.SemaphoreType.DMA,
            pltpu.SemaphoreType.DMA,
        ],
    )


def kernel(f0, phoneme_seq, singer_id, language_id, phoneme_table,
           singer_table, language_table, W, b):
    idx = phoneme_seq.astype(jnp.int32)
    sid = singer_id.astype(jnp.int32).reshape(B, 1)
    lid = language_id.astype(jnp.int32).reshape(B, 1)
    bias = b.reshape(1, NMEL)

    P, base = _precompute(phoneme_table, singer_table, language_table,
                          sid, lid, W, bias)

    out = _sc_lookup()(P.reshape(-1), W, base.reshape(-1),
                       f0.reshape(NW, BPW * T), idx.reshape(NW, BPW * T))
    return out.reshape(B, T, NMEL)
